# baseline (speedup 1.0000x reference)
import functools
import jax
import jax.numpy as jnp
from jax import lax
from jax.experimental import pallas as pl
from jax.experimental.pallas import tpu as pltpu
from jax.experimental.pallas import tpu_sc as plsc

D = 128
L = 16
NBUF = 3
K = 4
EPS2 = (1e-12) ** 2 / 128.0


def _rsqrt(ssv):
    i = plsc.bitcast(ssv, jnp.int32)
    y = plsc.bitcast(jnp.int32(0x5F3759DF) - (i >> 1), jnp.float32)
    y = y * (jnp.float32(1.5) - jnp.float32(0.5) * ssv * y * y)
    y = y * (jnp.float32(1.5) - jnp.float32(0.5) * ssv * y * y)
    return y


def _sumsq(row_ref, f):
    sq = [None] * (D // L)
    for j in range(D // L):
        x = row_ref[f, pl.ds(j * L, L)]
        sq[j] = x * x
    while len(sq) > 1:
        sq = [a + b for a, b in zip(sq[0::2], sq[1::2])]
    return sq[0]


def _scale_row(row_ref, f, iv):
    for j in range(D // L):
        row_ref[f, pl.ds(j * L, L)] = row_ref[f, pl.ds(j * L, L)] * iv


def _phase_b(sq_v, inv_v, q, n_groups):
    lane = lax.iota(jnp.int32, L)

    @plsc.parallel_loop(0, n_groups, 1, unroll=2)
    def groups_b(gi):
        rb = pl.multiple_of(gi * L, L)
        ts = []
        for j in range(L):
            ts.append(plsc.load_gather(
                sq_v, [rb + lane, jnp.full((L,), j, jnp.int32)]))
        while len(ts) > 1:
            ts = [a + b for a, b in zip(ts[0::2], ts[1::2])]
        inv_v[q, pl.ds(rb, L)] = _rsqrt(jnp.maximum(ts[0], jnp.float32(EPS2)))


def kernel(tokens, table):
    n_rows, row_len = tokens.shape
    info = plsc.get_sparse_core_info()
    NC, NS = info.num_cores, info.num_subcores
    NW = NC * NS
    rows_per_w = n_rows // NW
    n_chunks = rows_per_w // K
    n_flat = K * row_len
    n_groups = (n_flat + L - 1) // L

    mesh = plsc.VectorSubcoreMesh(core_axis_name="c", subcore_axis_name="s")

    @functools.partial(
        pl.kernel,
        mesh=mesh,
        compiler_params=pltpu.CompilerParams(needs_layout_passes=False),
        out_type=jax.ShapeDtypeStruct((n_rows, row_len, D), jnp.float32),
        scratch_types=[
            pltpu.VMEM((rows_per_w, row_len), jnp.int32),
            pltpu.VMEM((NBUF, n_flat, D), jnp.float32),
            pltpu.VMEM((n_groups * L, L + 1), jnp.float32),
            pltpu.VMEM((2, n_groups * L), jnp.float32),
            pltpu.SemaphoreType.DMA((NBUF,)),
            pltpu.SemaphoreType.DMA((NBUF,)),
        ],
    )
    def sc_embed(idx_hbm, table_hbm, out_hbm, idx_v, rows_v, sq_v, inv_v,
                 sem_in, sem_out):
        wid = lax.axis_index("s") * NC + lax.axis_index("c")
        base = wid * rows_per_w
        pltpu.sync_copy(idx_hbm.at[pl.ds(base, rows_per_w), :], idx_v)

        def gather_copy(g, b, k):
            return pltpu.make_async_copy(
                table_hbm.at[idx_v.at[g * K + k]],
                rows_v.at[b, pl.ds(k * row_len, row_len), :],
                sem_in.at[b])

        def out_copy(g, b, k):
            return pltpu.make_async_copy(
                rows_v.at[b, pl.ds(k * row_len, row_len), :],
                out_hbm.at[base + g * K + k],
                sem_out.at[b])

        def start_gathers(g, b):
            for k in range(K):
                gather_copy(g, b, k).start()

        def wait_gathers(g, b):
            for k in range(K):
                gather_copy(g, b, k).wait()

        def start_outs(g, b):
            for k in range(K):
                out_copy(g, b, k).start()

        def wait_outs(g, b):
            for k in range(K):
                out_copy(g, b, k).wait()

        start_gathers(0, 0)
        wait_gathers(0, 0)
        start_gathers(1, 1)

        @plsc.parallel_loop(0, n_flat, 1, unroll=8)
        def rows_a0(f):
            sq_v[f, pl.ds(0, L)] = _sumsq(rows_v.at[0], f)

        _phase_b(sq_v, inv_v, 0, n_groups)

        def chunk_body(g, _):
            b = lax.rem(g, NBUF)
            bp = lax.rem(g + NBUF - 1, NBUF)
            p = lax.rem(g + 1, 2)
            q = lax.rem(g, 2)
            wait_gathers(g, b)

            @pl.when(g >= 2)
            def _drain():
                wait_outs(g - 2, lax.rem(g + NBUF - 2, NBUF))

            @pl.when(g + 1 < n_chunks)
            def _prefetch():
                start_gathers(g + 1, lax.rem(g + 1, NBUF))

            @plsc.parallel_loop(0, n_flat, 1, unroll=8)
            def rows_ac(f):
                sq_v[f, pl.ds(0, L)] = _sumsq(rows_v.at[b], f)
                iv = plsc.load_gather(
                    inv_v, [jnp.full((L,), 0, jnp.int32) + p,
                            jnp.full((L,), 0, jnp.int32) + f])
                _scale_row(rows_v.at[bp], f, iv)

            start_outs(g - 1, bp)
            _phase_b(sq_v, inv_v, q, n_groups)
            return _

        lax.fori_loop(1, n_chunks, chunk_body, None)

        gl = n_chunks - 1
        bl = gl % NBUF
        pq = gl % 2

        @plsc.parallel_loop(0, n_flat, 1, unroll=8)
        def rows_cl(f):
            iv = plsc.load_gather(
                inv_v, [jnp.full((L,), pq, jnp.int32),
                        jnp.full((L,), 0, jnp.int32) + f])
            _scale_row(rows_v.at[bl], f, iv)

        start_outs(gl, bl)
        wait_outs(gl - 1, (gl - 1) % NBUF)
        wait_outs(gl, bl)

    return sc_embed(tokens.astype(jnp.int32), table)
